# baseline (speedup 1.0000x reference)
import functools

import jax
import jax.numpy as jnp
from jax import lax
from jax.experimental import pallas as pl
from jax.experimental.pallas import tpu as pltpu
from jax.experimental.pallas import tpu_sc as plsc

_NW = 32
_LANES = 16
_ROWLANES = 128
_CHR = 64


def _sc_partials_body(x_hbm, p_hbm, n_hbm, out_hbm, xb, pb, nb, accv, sem0, sem1):
    nc = 2
    w = lax.axis_index("s") * nc + lax.axis_index("c")
    rows_total = x_hbm.shape[0]
    rows_per_w = rows_total // _NW
    n_chunks = rows_per_w // _CHR
    base = w * rows_per_w
    b = w // 2
    h = w % 2

    sems = (sem0, sem1)
    bufs = ((xb, x_hbm), (pb, p_hbm), (nb, n_hbm))

    def start(ci):
        slot = ci % 2
        off = base + ci * _CHR
        return [
            pltpu.async_copy(hbm.at[pl.ds(off, _CHR)], buf.at[slot], sems[slot])
            for (buf, hbm) in bufs
        ]

    inflight = {0: start(0)}

    acc_s = jnp.zeros((_LANES,), jnp.float32)
    acc_c = jnp.zeros((_LANES,), jnp.float32)

    for ci in range(n_chunks):
        if ci + 1 < n_chunks:
            inflight[ci + 1] = start(ci + 1)
        for hdl in inflight.pop(ci):
            hdl.wait()
        slot = ci % 2

        def body(i, carry):
            a_s, a_c = carry
            for u in range(_ROWLANES // _LANES):
                xv = xb[slot, i, pl.ds(u * _LANES, _LANES)]
                pv = pb[slot, i, pl.ds(u * _LANES, _LANES)]
                nv = nb[slot, i, pl.ds(u * _LANES, _LANES)]
                mf = jnp.where(jnp.minimum(pv, nv) > 0.5, 1.0, 0.0)
                probs = 1.0 / (1.0 + jnp.exp(-xv))
                a_s = a_s + probs * mf
                a_c = a_c + mf
            return a_s, a_c

        acc_s, acc_c = lax.fori_loop(0, _CHR, body, (acc_s, acc_c))

    accv[0, :] = acc_s
    accv[1, :] = acc_c
    pltpu.sync_copy(accv, out_hbm.at[h, b])


def _make_sc_partials(batches):
    return functools.partial(
        pl.kernel,
        out_type=jax.ShapeDtypeStruct((2, batches, 2, _LANES), jnp.float32),
        mesh=plsc.VectorSubcoreMesh(core_axis_name="c", subcore_axis_name="s"),
        scratch_types=[
            pltpu.VMEM((2, _CHR, _ROWLANES), jnp.float32),
            pltpu.VMEM((2, _CHR, _ROWLANES), jnp.float32),
            pltpu.VMEM((2, _CHR, _ROWLANES), jnp.float32),
            pltpu.VMEM((2, _LANES), jnp.float32),
            pltpu.SemaphoreType.DMA,
            pltpu.SemaphoreType.DMA,
        ],
    )(_sc_partials_body)


def _combine_body(inv_ref, part_ref, out_ref):
    part = part_ref[...]
    a = part[0] + part[1]
    red = jnp.sum(a, axis=2)
    p = red[:, 0:1] / red[:, 1:2]
    inv = inv_ref[...]
    terms = -inv * jnp.log(p) - (1.0 - inv) * jnp.log(1.0 - p)
    out_ref[...] = jnp.sum(terms).reshape(1, 1)


def kernel(cancer_logits, prostate_mask, needle_mask, involvement, grade_group):
    B, _, H, W = cancer_logits.shape
    x = cancer_logits.reshape(B * H * W // _ROWLANES, _ROWLANES)
    pm = prostate_mask.reshape(B * H * W // _ROWLANES, _ROWLANES)
    nm = needle_mask.reshape(B * H * W // _ROWLANES, _ROWLANES)

    part = _make_sc_partials(B)(x, pm, nm)

    out = pl.pallas_call(
        _combine_body,
        in_specs=[
            pl.BlockSpec(memory_space=pltpu.VMEM),
            pl.BlockSpec(memory_space=pltpu.VMEM),
        ],
        out_specs=pl.BlockSpec(memory_space=pltpu.VMEM),
        out_shape=jax.ShapeDtypeStruct((1, 1), jnp.float32),
    )(involvement.reshape(B, 1), part)
    return out[0, 0]


# speedup vs baseline: 1.3326x; 1.3326x over previous
import functools

import jax
import jax.numpy as jnp
from jax import lax
from jax.experimental import pallas as pl
from jax.experimental.pallas import tpu as pltpu
from jax.experimental.pallas import tpu_sc as plsc

_NW = 32
_LANES = 16
_ROWLANES = 512
_CHR = 16


def _sc_partials_body(x_hbm, p_hbm, n_hbm, out_hbm, xb, pb, nb, accv, sem0, sem1):
    nc = 2
    w = lax.axis_index("s") * nc + lax.axis_index("c")
    rows_total = x_hbm.shape[0]
    rows_per_w = rows_total // _NW
    n_chunks = rows_per_w // _CHR
    base = w * rows_per_w
    b = w // 2
    h = w % 2

    sems = (sem0, sem1)
    bufs = ((xb, x_hbm), (pb, p_hbm), (nb, n_hbm))

    def start(ci):
        slot = ci % 2
        off = base + ci * _CHR
        return [
            pltpu.async_copy(hbm.at[pl.ds(off, _CHR)], buf.at[slot], sems[slot])
            for (buf, hbm) in bufs
        ]

    inflight = {0: start(0)}

    acc_s = jnp.zeros((_LANES,), jnp.float32)
    acc_c = jnp.zeros((_LANES,), jnp.float32)

    for ci in range(n_chunks):
        if ci + 1 < n_chunks:
            inflight[ci + 1] = start(ci + 1)
        for hdl in inflight.pop(ci):
            hdl.wait()
        slot = ci % 2

        def body(i, carry):
            a_s, a_c = carry
            for u in range(_ROWLANES // _LANES):
                xv = xb[slot, i, pl.ds(u * _LANES, _LANES)]
                pv = pb[slot, i, pl.ds(u * _LANES, _LANES)]
                nv = nb[slot, i, pl.ds(u * _LANES, _LANES)]
                mf = jnp.where(jnp.minimum(pv, nv) > 0.5, 1.0, 0.0)
                probs = 1.0 / (1.0 + jnp.exp(-xv))
                a_s = a_s + probs * mf
                a_c = a_c + mf
            return a_s, a_c

        acc_s, acc_c = lax.fori_loop(0, _CHR, body, (acc_s, acc_c))

    accv[0, :] = acc_s
    accv[1, :] = acc_c
    pltpu.sync_copy(accv, out_hbm.at[h, b])


def _make_sc_partials(batches):
    return functools.partial(
        pl.kernel,
        out_type=jax.ShapeDtypeStruct((2, batches, 2, _LANES), jnp.float32),
        mesh=plsc.VectorSubcoreMesh(core_axis_name="c", subcore_axis_name="s"),
        scratch_types=[
            pltpu.VMEM((2, _CHR, _ROWLANES), jnp.float32),
            pltpu.VMEM((2, _CHR, _ROWLANES), jnp.float32),
            pltpu.VMEM((2, _CHR, _ROWLANES), jnp.float32),
            pltpu.VMEM((2, _LANES), jnp.float32),
            pltpu.SemaphoreType.DMA,
            pltpu.SemaphoreType.DMA,
        ],
    )(_sc_partials_body)


def _combine_body(inv_ref, part_ref, out_ref):
    part = part_ref[...]
    a = part[0] + part[1]
    red = jnp.sum(a, axis=2)
    p = red[:, 0:1] / red[:, 1:2]
    inv = inv_ref[...]
    terms = -inv * jnp.log(p) - (1.0 - inv) * jnp.log(1.0 - p)
    out_ref[...] = jnp.sum(terms).reshape(1, 1)


def kernel(cancer_logits, prostate_mask, needle_mask, involvement, grade_group):
    B, _, H, W = cancer_logits.shape
    x = cancer_logits.reshape(B * H * W // _ROWLANES, _ROWLANES)
    pm = prostate_mask.reshape(B * H * W // _ROWLANES, _ROWLANES)
    nm = needle_mask.reshape(B * H * W // _ROWLANES, _ROWLANES)

    part = _make_sc_partials(B)(x, pm, nm)

    out = pl.pallas_call(
        _combine_body,
        in_specs=[
            pl.BlockSpec(memory_space=pltpu.VMEM),
            pl.BlockSpec(memory_space=pltpu.VMEM),
        ],
        out_specs=pl.BlockSpec(memory_space=pltpu.VMEM),
        out_shape=jax.ShapeDtypeStruct((1, 1), jnp.float32),
    )(involvement.reshape(B, 1), part)
    return out[0, 0]
